# Initial kernel scaffold; baseline (speedup 1.0000x reference)
import jax
import jax.numpy as jnp
from jax.experimental import pallas as pl


def kernel(query, keys, W, gamma, beta):
    raise NotImplementedError("write your pallas kernel here")


# speedup vs baseline: 6.5705x; 6.5705x over previous
import functools

import jax
import jax.numpy as jnp
from jax import lax
from jax.experimental import pallas as pl

_N = 8192
_M = 1024
_D = 256
_B = 8
_HW = 1024
_MB = 128


def _phase_a(q_ref, keys_ref, w_ref,
             qn_ref, sm_ref, conv_ref, bnsum_ref, bnsq_ref, gp_ref, sp_ref):
    f32 = jnp.float32
    q = q_ref[...].reshape(_D, _HW)
    n2 = jnp.sum(q * q, axis=0, keepdims=True)
    qn = q / jnp.maximum(jnp.sqrt(n2), 1e-12)
    qn_ref[...] = qn

    keys = keys_ref[...]
    s = lax.dot_general(qn, keys, (((0,), (1,)), ((), ())),
                        preferred_element_type=f32)

    m1 = jnp.max(s, axis=1, keepdims=True)
    e = jnp.exp(s - m1)
    p = e / jnp.sum(e, axis=1, keepdims=True)
    sm_ref[...] = p

    iota = lax.broadcasted_iota(jnp.int32, (_HW, _M), 1)
    big = jnp.int32(2**30)
    i1 = jnp.min(jnp.where(s == m1, iota, big), axis=1, keepdims=True)
    s2 = jnp.where(iota == i1, -jnp.inf, s)
    m2 = jnp.max(s2, axis=1, keepdims=True)
    i2 = jnp.min(jnp.where(s2 == m2, iota, big), axis=1, keepdims=True)

    kstats = jnp.concatenate(
        [jnp.sum(keys * keys, axis=1, keepdims=True),
         jnp.sum(keys, axis=1, keepdims=True)], axis=1)
    hi = jax.lax.Precision.HIGHEST
    oh1 = (iota == i1).astype(f32)
    oh2 = (iota == i2).astype(f32)
    k1 = lax.dot_general(oh1, kstats, (((1,), (0,)), ((), ())),
                         precision=hi, preferred_element_type=f32)
    k2 = lax.dot_general(oh2, kstats, (((1,), (0,)), ((), ())),
                         precision=hi, preferred_element_type=f32)
    ones = jnp.ones((_D, 1), f32)
    qsq = lax.dot_general(qn * qn, ones, (((0,), (0,)), ((), ())),
                          precision=hi, preferred_element_type=f32)
    qsum = lax.dot_general(qn, ones, (((0,), (0,)), ((), ())),
                           precision=hi, preferred_element_type=f32)

    eps = 1e-6
    epsterm = _D * eps * eps
    gterm = qsq - 2.0 * m1 + k1[:, 0:1]
    dpos = jnp.sqrt(jnp.maximum(
        gterm + 2.0 * eps * (qsum - k1[:, 1:2]) + epsterm, 0.0))
    dneg = jnp.sqrt(jnp.maximum(
        qsq - 2.0 * m2 + k2[:, 0:1] + 2.0 * eps * (qsum - k2[:, 1:2]) + epsterm,
        0.0))
    gp = jnp.sum(gterm)
    sp = jnp.sum(jnp.maximum(dpos - dneg + 1.0, 0.0))
    gp_ref[...] = jnp.full((1, 1, 128), gp, f32)
    sp_ref[...] = jnp.full((1, 1, 128), sp, f32)

    c = lax.dot_general(p, keys, (((1,), (0,)), ((), ())),
                        preferred_element_type=f32)
    w1 = w_ref[0:_D, :]
    w2 = w_ref[_D:2 * _D, :]
    conv = (lax.dot_general(qn, w1, (((0,), (0,)), ((), ())),
                            preferred_element_type=f32) +
            lax.dot_general(c, w2, (((1,), (0,)), ((), ())),
                            preferred_element_type=f32))
    conv_ref[...] = conv
    bnsum_ref[...] = jnp.sum(conv, axis=0, keepdims=True).reshape(1, 1, _D)
    bnsq_ref[...] = jnp.sum(conv * conv, axis=0, keepdims=True).reshape(1, 1, _D)


def _phase_b(qn_ref, keysb_ref, conv_ref, bnsum_ref, bnsq_ref,
             gamma_ref, beta_ref, gp_ref, sp_ref,
             sq_ref, uq_ref, gl_ref, sl_ref):
    f32 = jnp.float32
    qn = qn_ref[...]
    kb = keysb_ref[...]
    s = lax.dot_general(qn, kb, (((0,), (1,)), ((), ())),
                        preferred_element_type=f32)
    cm = jnp.max(s, axis=0, keepdims=True)
    e = jnp.exp(s - cm)
    sq_ref[...] = e / jnp.sum(e, axis=0, keepdims=True)

    conv = conv_ref[...]
    bnsum = jnp.sum(bnsum_ref[...].reshape(_B, _D), axis=0, keepdims=True)
    bnsq = jnp.sum(bnsq_ref[...].reshape(_B, _D), axis=0, keepdims=True)
    mean = bnsum * (1.0 / _N)
    var = bnsq * (1.0 / _N) - mean * mean
    inv = 1.0 / jnp.sqrt(var + 1e-5)
    y = jnp.maximum((conv - mean) * inv * gamma_ref[...] + beta_ref[...], 0.0)
    uq_ref[...] = jnp.transpose(y, (1, 0)).reshape(1, _D, 32, 32)

    gl_ref[...] = jnp.sum(gp_ref[...][:, 0, 0]).reshape(1, 1) * (1.0 / (_N * _D))
    sl_ref[...] = jnp.sum(sp_ref[...][:, 0, 0]).reshape(1, 1) * (1.0 / _N)


@functools.partial(jax.jit, static_argnames=())
def kernel(query, keys, W, gamma, beta):
    f32 = jnp.float32
    qn, sm, conv, bnsum, bnsq, gp, sp = pl.pallas_call(
        _phase_a,
        grid=(_B,),
        in_specs=[
            pl.BlockSpec((1, _D, 32, 32), lambda i: (i, 0, 0, 0)),
            pl.BlockSpec((_M, _D), lambda i: (0, 0)),
            pl.BlockSpec((2 * _D, _D), lambda i: (0, 0)),
        ],
        out_specs=[
            pl.BlockSpec((_D, _HW), lambda i: (0, i)),
            pl.BlockSpec((_HW, _M), lambda i: (i, 0)),
            pl.BlockSpec((_HW, _D), lambda i: (i, 0)),
            pl.BlockSpec((1, 1, _D), lambda i: (i, 0, 0)),
            pl.BlockSpec((1, 1, _D), lambda i: (i, 0, 0)),
            pl.BlockSpec((1, 1, 128), lambda i: (i, 0, 0)),
            pl.BlockSpec((1, 1, 128), lambda i: (i, 0, 0)),
        ],
        out_shape=[
            jax.ShapeDtypeStruct((_D, _N), f32),
            jax.ShapeDtypeStruct((_N, _M), f32),
            jax.ShapeDtypeStruct((_N, _D), f32),
            jax.ShapeDtypeStruct((_B, 1, _D), f32),
            jax.ShapeDtypeStruct((_B, 1, _D), f32),
            jax.ShapeDtypeStruct((_B, 1, 128), f32),
            jax.ShapeDtypeStruct((_B, 1, 128), f32),
        ],
    )(query, keys, W)

    sq, uq, gl, sl = pl.pallas_call(
        _phase_b,
        grid=(_B,),
        in_specs=[
            pl.BlockSpec((_D, _N), lambda j: (0, 0)),
            pl.BlockSpec((_MB, _D), lambda j: (j, 0)),
            pl.BlockSpec((_HW, _D), lambda j: (j, 0)),
            pl.BlockSpec((_B, 1, _D), lambda j: (0, 0, 0)),
            pl.BlockSpec((_B, 1, _D), lambda j: (0, 0, 0)),
            pl.BlockSpec((1, _D), lambda j: (0, 0)),
            pl.BlockSpec((1, _D), lambda j: (0, 0)),
            pl.BlockSpec((_B, 1, 128), lambda j: (0, 0, 0)),
            pl.BlockSpec((_B, 1, 128), lambda j: (0, 0, 0)),
        ],
        out_specs=[
            pl.BlockSpec((_N, _MB), lambda j: (0, j)),
            pl.BlockSpec((1, _D, 32, 32), lambda j: (j, 0, 0, 0)),
            pl.BlockSpec((1, 1), lambda j: (0, 0)),
            pl.BlockSpec((1, 1), lambda j: (0, 0)),
        ],
        out_shape=[
            jax.ShapeDtypeStruct((_N, _M), f32),
            jax.ShapeDtypeStruct((_B, _D, 32, 32), f32),
            jax.ShapeDtypeStruct((1, 1), f32),
            jax.ShapeDtypeStruct((1, 1), f32),
        ],
    )(qn, keys, conv, bnsum, bnsq,
      gamma.reshape(1, _D), beta.reshape(1, _D), gp, sp)

    return (uq, sq, sm, gl.reshape(()), sl.reshape(()))


# speedup vs baseline: 8.3641x; 1.2730x over previous
import functools

import jax
import jax.numpy as jnp
from jax import lax
from jax.experimental import pallas as pl

_N = 8192
_M = 1024
_D = 256
_B = 8
_HW = 1024
_MB = 128


def _phase_a(q_ref, keys_ref, w_ref,
             qn_ref, sm_ref, conv_ref, bnsum_ref, bnsq_ref, gp_ref, sp_ref):
    f32 = jnp.float32
    q = q_ref[...].reshape(_D, _HW)
    n2 = jnp.sum(q * q, axis=0, keepdims=True)
    denom = jnp.maximum(jnp.sqrt(n2), 1e-12)
    qn = q / denom
    qn_ref[...] = qn
    qsq_l = n2 / (denom * denom)

    keys = keys_ref[...]
    s = lax.dot_general(qn, keys, (((0,), (1,)), ((), ())),
                        preferred_element_type=f32)

    m1 = jnp.max(s, axis=1, keepdims=True)
    e = jnp.exp(s - m1)
    p = e / jnp.sum(e, axis=1, keepdims=True)
    sm_ref[...] = p

    iota = lax.broadcasted_iota(jnp.int32, (_HW, _M), 1)
    big = jnp.int32(2**30)
    i1 = jnp.min(jnp.where(s == m1, iota, big), axis=1, keepdims=True)
    mask1 = iota == i1
    s2 = jnp.where(mask1, -jnp.inf, s)
    m2 = jnp.max(s2, axis=1, keepdims=True)
    i2 = jnp.min(jnp.where(s2 == m2, iota, big), axis=1, keepdims=True)
    mask2 = iota == i2

    hi = jax.lax.Precision.HIGHEST
    eps = 1e-6
    ones_d = jnp.ones((1, _D), f32)
    ksq_t = lax.dot_general(ones_d, keys * keys, (((1,), (1,)), ((), ())),
                            precision=hi, preferred_element_type=f32)
    ksum_t = lax.dot_general(ones_d, keys, (((1,), (1,)), ((), ())),
                             precision=hi, preferred_element_type=f32)
    g_t = ksq_t - (2.0 * eps) * ksum_t
    kv1 = jnp.sum(jnp.where(mask1, g_t, 0.0), axis=1, keepdims=True)
    kv2 = jnp.sum(jnp.where(mask2, g_t, 0.0), axis=1, keepdims=True)
    ones_c = jnp.ones((_D, 1), f32)
    a = lax.dot_general(qn * qn + (2.0 * eps) * qn, ones_c,
                        (((0,), (0,)), ((), ())),
                        precision=hi, preferred_element_type=f32)
    a = a + _D * eps * eps

    dpos = jnp.sqrt(jnp.maximum(a - 2.0 * m1 + kv1, 0.0))
    dneg = jnp.sqrt(jnp.maximum(a - 2.0 * m2 + kv2, 0.0))
    cnt1 = jnp.sum(mask1.astype(f32), axis=0, keepdims=True)
    gp = (jnp.sum(qsq_l) - 2.0 * jnp.sum(m1) + jnp.sum(ksq_t * cnt1))
    sp = jnp.sum(jnp.maximum(dpos - dneg + 1.0, 0.0))
    gp_ref[...] = jnp.full((1, 1, 128), gp, f32)
    sp_ref[...] = jnp.full((1, 1, 128), sp, f32)

    c = lax.dot_general(p, keys, (((1,), (0,)), ((), ())),
                        preferred_element_type=f32)
    w1 = w_ref[0:_D, :]
    w2 = w_ref[_D:2 * _D, :]
    conv = (lax.dot_general(qn, w1, (((0,), (0,)), ((), ())),
                            preferred_element_type=f32) +
            lax.dot_general(c, w2, (((1,), (0,)), ((), ())),
                            preferred_element_type=f32))
    conv_ref[...] = conv
    bnsum_ref[...] = jnp.sum(conv, axis=0, keepdims=True).reshape(1, 1, _D)
    bnsq_ref[...] = jnp.sum(conv * conv, axis=0, keepdims=True).reshape(1, 1, _D)


def _phase_b(qn_ref, keysb_ref, conv_ref, bnsum_ref, bnsq_ref,
             gamma_ref, beta_ref, gp_ref, sp_ref,
             sq_ref, uq_ref, gl_ref, sl_ref):
    f32 = jnp.float32
    qn = qn_ref[...]
    kb = keysb_ref[...]
    s = lax.dot_general(qn, kb, (((0,), (1,)), ((), ())),
                        preferred_element_type=f32)
    cm = jnp.max(s, axis=0, keepdims=True)
    e = jnp.exp(s - cm)
    sq_ref[...] = e / jnp.sum(e, axis=0, keepdims=True)

    conv = conv_ref[...]
    bnsum = jnp.sum(bnsum_ref[...].reshape(_B, _D), axis=0, keepdims=True)
    bnsq = jnp.sum(bnsq_ref[...].reshape(_B, _D), axis=0, keepdims=True)
    mean = bnsum * (1.0 / _N)
    var = bnsq * (1.0 / _N) - mean * mean
    inv = 1.0 / jnp.sqrt(var + 1e-5)
    y = jnp.maximum((conv - mean) * inv * gamma_ref[...] + beta_ref[...], 0.0)
    uq_ref[...] = jnp.transpose(y, (1, 0)).reshape(1, _D, 32, 32)

    gl_ref[...] = jnp.sum(gp_ref[...][:, 0, 0]).reshape(1, 1) * (1.0 / (_N * _D))
    sl_ref[...] = jnp.sum(sp_ref[...][:, 0, 0]).reshape(1, 1) * (1.0 / _N)


@functools.partial(jax.jit, static_argnames=())
def kernel(query, keys, W, gamma, beta):
    f32 = jnp.float32
    qn, sm, conv, bnsum, bnsq, gp, sp = pl.pallas_call(
        _phase_a,
        grid=(_B,),
        in_specs=[
            pl.BlockSpec((1, _D, 32, 32), lambda i: (i, 0, 0, 0)),
            pl.BlockSpec((_M, _D), lambda i: (0, 0)),
            pl.BlockSpec((2 * _D, _D), lambda i: (0, 0)),
        ],
        out_specs=[
            pl.BlockSpec((_D, _HW), lambda i: (0, i)),
            pl.BlockSpec((_HW, _M), lambda i: (i, 0)),
            pl.BlockSpec((_HW, _D), lambda i: (i, 0)),
            pl.BlockSpec((1, 1, _D), lambda i: (i, 0, 0)),
            pl.BlockSpec((1, 1, _D), lambda i: (i, 0, 0)),
            pl.BlockSpec((1, 1, 128), lambda i: (i, 0, 0)),
            pl.BlockSpec((1, 1, 128), lambda i: (i, 0, 0)),
        ],
        out_shape=[
            jax.ShapeDtypeStruct((_D, _N), f32),
            jax.ShapeDtypeStruct((_N, _M), f32),
            jax.ShapeDtypeStruct((_N, _D), f32),
            jax.ShapeDtypeStruct((_B, 1, _D), f32),
            jax.ShapeDtypeStruct((_B, 1, _D), f32),
            jax.ShapeDtypeStruct((_B, 1, 128), f32),
            jax.ShapeDtypeStruct((_B, 1, 128), f32),
        ],
    )(query, keys, W)

    sq, uq, gl, sl = pl.pallas_call(
        _phase_b,
        grid=(_B,),
        in_specs=[
            pl.BlockSpec((_D, _N), lambda j: (0, 0)),
            pl.BlockSpec((_MB, _D), lambda j: (j, 0)),
            pl.BlockSpec((_HW, _D), lambda j: (j, 0)),
            pl.BlockSpec((_B, 1, _D), lambda j: (0, 0, 0)),
            pl.BlockSpec((_B, 1, _D), lambda j: (0, 0, 0)),
            pl.BlockSpec((1, _D), lambda j: (0, 0)),
            pl.BlockSpec((1, _D), lambda j: (0, 0)),
            pl.BlockSpec((_B, 1, 128), lambda j: (0, 0, 0)),
            pl.BlockSpec((_B, 1, 128), lambda j: (0, 0, 0)),
        ],
        out_specs=[
            pl.BlockSpec((_N, _MB), lambda j: (0, j)),
            pl.BlockSpec((1, _D, 32, 32), lambda j: (j, 0, 0, 0)),
            pl.BlockSpec((1, 1), lambda j: (0, 0)),
            pl.BlockSpec((1, 1), lambda j: (0, 0)),
        ],
        out_shape=[
            jax.ShapeDtypeStruct((_N, _M), f32),
            jax.ShapeDtypeStruct((_B, _D, 32, 32), f32),
            jax.ShapeDtypeStruct((1, 1), f32),
            jax.ShapeDtypeStruct((1, 1), f32),
        ],
    )(qn, keys, conv, bnsum, bnsq,
      gamma.reshape(1, _D), beta.reshape(1, _D), gp, sp)

    return (uq, sq, sm, gl.reshape(()), sl.reshape(()))


# speedup vs baseline: 12.6459x; 1.5119x over previous
import functools

import jax
import jax.numpy as jnp
from jax import lax
from jax.experimental import pallas as pl

_N = 8192
_M = 1024
_D = 256
_B = 8
_HW = 1024
_MB = 128


def _phase_a(q_ref, keys_ref, w_ref,
             qn_ref, sm_ref, conv_ref, bnsum_ref, bnsq_ref, gp_ref, sp_ref):
    f32 = jnp.float32
    q = q_ref[...].reshape(_D, _HW)
    n2 = jnp.sum(q * q, axis=0, keepdims=True)
    denom = jnp.maximum(jnp.sqrt(n2), 1e-12)
    qn = q / denom
    qn_ref[...] = qn
    qsq_l = n2 / (denom * denom)
    qsum_l = jnp.sum(q, axis=0, keepdims=True) / denom

    eps = 1e-6
    a_l = qsq_l + (2.0 * eps) * qsum_l + _D * eps * eps
    a = jnp.transpose(a_l, (1, 0))

    keys = keys_ref[...]
    s = lax.dot_general(qn, keys, (((0,), (1,)), ((), ())),
                        preferred_element_type=f32)

    m1 = jnp.max(s, axis=1, keepdims=True)
    e = jnp.exp(s - m1)
    p = e * (1.0 / jnp.sum(e, axis=1, keepdims=True))
    sm_ref[...] = p

    hi = jax.lax.Precision.HIGHEST
    ones_d = jnp.ones((1, _D), f32)
    ksq_t = lax.dot_general(ones_d, keys * keys, (((1,), (1,)), ((), ())),
                            precision=hi, preferred_element_type=f32)
    g_t = lax.dot_general(ones_d, keys * (keys - 2.0 * eps),
                          (((1,), (1,)), ((), ())),
                          precision=hi, preferred_element_type=f32)

    f1 = (s == m1).astype(f32)
    r1 = 1.0 / jnp.sum(f1, axis=1, keepdims=True)
    kv1 = jnp.sum(f1 * g_t, axis=1, keepdims=True) * r1
    ksq1 = jnp.sum(f1 * ksq_t, axis=1, keepdims=True) * r1
    s2 = jnp.where(s == m1, -jnp.inf, s)
    m2 = jnp.max(s2, axis=1, keepdims=True)
    f2 = (s2 == m2).astype(f32)
    r2 = 1.0 / jnp.sum(f2, axis=1, keepdims=True)
    kv2 = jnp.sum(f2 * g_t, axis=1, keepdims=True) * r2

    dpos = jnp.sqrt(jnp.maximum(a - 2.0 * m1 + kv1, 0.0))
    dneg = jnp.sqrt(jnp.maximum(a - 2.0 * m2 + kv2, 0.0))
    gp = jnp.sum(qsq_l) - 2.0 * jnp.sum(m1) + jnp.sum(ksq1)
    sp = jnp.sum(jnp.maximum(dpos - dneg + 1.0, 0.0))
    gp_ref[...] = jnp.full((1, 1, 128), gp, f32)
    sp_ref[...] = jnp.full((1, 1, 128), sp, f32)

    c_t = lax.dot_general(keys, p, (((0,), (1,)), ((), ())),
                          preferred_element_type=f32)
    w1 = w_ref[0:_D, :]
    w2 = w_ref[_D:2 * _D, :]
    conv = (lax.dot_general(w1, qn, (((0,), (0,)), ((), ())),
                            preferred_element_type=f32) +
            lax.dot_general(w2, c_t, (((0,), (0,)), ((), ())),
                            preferred_element_type=f32))
    conv_ref[...] = conv
    bnsum_ref[...] = jnp.sum(conv, axis=1, keepdims=True).reshape(1, _D, 1)
    bnsq_ref[...] = jnp.sum(conv * conv, axis=1, keepdims=True).reshape(1, _D, 1)


def _phase_b(qn_ref, keysb_ref, conv_ref, bnsum_ref, bnsq_ref,
             gamma_ref, beta_ref, gp_ref, sp_ref,
             sq_ref, uq_ref, gl_ref, sl_ref):
    f32 = jnp.float32
    qn = qn_ref[...]
    kb = keysb_ref[...]
    s = lax.dot_general(qn, kb, (((0,), (1,)), ((), ())),
                        preferred_element_type=f32)
    cm = jnp.max(s, axis=0, keepdims=True)
    e = jnp.exp(s - cm)
    sq_ref[...] = e * (1.0 / jnp.sum(e, axis=0, keepdims=True))

    conv = conv_ref[...]
    mean = jnp.sum(bnsum_ref[...], axis=0) * (1.0 / _N)
    var = jnp.sum(bnsq_ref[...], axis=0) * (1.0 / _N) - mean * mean
    inv = 1.0 / jnp.sqrt(var + 1e-5)
    y = jnp.maximum((conv - mean) * inv * gamma_ref[...] + beta_ref[...], 0.0)
    uq_ref[...] = y.reshape(1, _D, _HW)

    gl_ref[...] = jnp.sum(gp_ref[...][:, 0, 0]).reshape(1, 1) * (1.0 / (_N * _D))
    sl_ref[...] = jnp.sum(sp_ref[...][:, 0, 0]).reshape(1, 1) * (1.0 / _N)


@functools.partial(jax.jit, static_argnames=())
def kernel(query, keys, W, gamma, beta):
    f32 = jnp.float32
    q3 = query.reshape(_B, _D, _HW)
    qn, sm, conv, bnsum, bnsq, gp, sp = pl.pallas_call(
        _phase_a,
        grid=(_B,),
        in_specs=[
            pl.BlockSpec((1, _D, _HW), lambda i: (i, 0, 0)),
            pl.BlockSpec((_M, _D), lambda i: (0, 0)),
            pl.BlockSpec((2 * _D, _D), lambda i: (0, 0)),
        ],
        out_specs=[
            pl.BlockSpec((_D, _HW), lambda i: (0, i)),
            pl.BlockSpec((_HW, _M), lambda i: (i, 0)),
            pl.BlockSpec((_D, _HW), lambda i: (0, i)),
            pl.BlockSpec((1, _D, 1), lambda i: (i, 0, 0)),
            pl.BlockSpec((1, _D, 1), lambda i: (i, 0, 0)),
            pl.BlockSpec((1, 1, 128), lambda i: (i, 0, 0)),
            pl.BlockSpec((1, 1, 128), lambda i: (i, 0, 0)),
        ],
        out_shape=[
            jax.ShapeDtypeStruct((_D, _N), f32),
            jax.ShapeDtypeStruct((_N, _M), f32),
            jax.ShapeDtypeStruct((_D, _N), f32),
            jax.ShapeDtypeStruct((_B, _D, 1), f32),
            jax.ShapeDtypeStruct((_B, _D, 1), f32),
            jax.ShapeDtypeStruct((_B, 1, 128), f32),
            jax.ShapeDtypeStruct((_B, 1, 128), f32),
        ],
    )(q3, keys, W)

    sq, uq, gl, sl = pl.pallas_call(
        _phase_b,
        grid=(_B,),
        in_specs=[
            pl.BlockSpec((_D, _N), lambda j: (0, 0)),
            pl.BlockSpec((_MB, _D), lambda j: (j, 0)),
            pl.BlockSpec((_D, _HW), lambda j: (0, j)),
            pl.BlockSpec((_B, _D, 1), lambda j: (0, 0, 0)),
            pl.BlockSpec((_B, _D, 1), lambda j: (0, 0, 0)),
            pl.BlockSpec((_D, 1), lambda j: (0, 0)),
            pl.BlockSpec((_D, 1), lambda j: (0, 0)),
            pl.BlockSpec((_B, 1, 128), lambda j: (0, 0, 0)),
            pl.BlockSpec((_B, 1, 128), lambda j: (0, 0, 0)),
        ],
        out_specs=[
            pl.BlockSpec((_N, _MB), lambda j: (0, j)),
            pl.BlockSpec((1, _D, _HW), lambda j: (j, 0, 0)),
            pl.BlockSpec((1, 1), lambda j: (0, 0)),
            pl.BlockSpec((1, 1), lambda j: (0, 0)),
        ],
        out_shape=[
            jax.ShapeDtypeStruct((_N, _M), f32),
            jax.ShapeDtypeStruct((_B, _D, _HW), f32),
            jax.ShapeDtypeStruct((1, 1), f32),
            jax.ShapeDtypeStruct((1, 1), f32),
        ],
    )(qn, keys, conv, bnsum, bnsq,
      gamma.reshape(_D, 1), beta.reshape(_D, 1), gp, sp)

    return (uq.reshape(_B, _D, 32, 32), sq, sm, gl.reshape(()), sl.reshape(()))


# speedup vs baseline: 13.4275x; 1.0618x over previous
import functools

import jax
import jax.numpy as jnp
from jax import lax
from jax.experimental import pallas as pl
from jax.experimental.pallas import tpu as pltpu

_N = 8192
_M = 1024
_D = 256
_B = 8
_HW = 1024
_MB = 128


def _fused(q_ref, keysf_ref, keysb_ref, w_ref, gamma_ref, beta_ref,
           sm_ref, sq_ref, uq_ref, gl_ref, sl_ref,
           qn_s, conv_s, bnsum_s, bnsq_s, gp_s, sp_s):
    f32 = jnp.float32
    i = pl.program_id(0)

    @pl.when(i < _B)
    def _phase_a():
        q = q_ref[...].reshape(_D, _HW)
        n2 = jnp.sum(q * q, axis=0, keepdims=True)
        denom = jnp.maximum(jnp.sqrt(n2), 1e-12)
        qn = q / denom
        qn_s[i] = qn
        qsq_l = n2 / (denom * denom)
        qsum_l = jnp.sum(q, axis=0, keepdims=True) / denom

        eps = 1e-6
        a_l = qsq_l + (2.0 * eps) * qsum_l + _D * eps * eps
        a = jnp.transpose(a_l, (1, 0))

        keys = keysf_ref[...]
        s = lax.dot_general(qn, keys, (((0,), (1,)), ((), ())),
                            preferred_element_type=f32)

        m1 = jnp.max(s, axis=1, keepdims=True)
        e = jnp.exp(s - m1)
        p = e * (1.0 / jnp.sum(e, axis=1, keepdims=True))
        sm_ref[...] = p

        hi = jax.lax.Precision.HIGHEST
        ones_d = jnp.ones((1, _D), f32)
        ksq_t = lax.dot_general(ones_d, keys * keys, (((1,), (1,)), ((), ())),
                                precision=hi, preferred_element_type=f32)
        g_t = lax.dot_general(ones_d, keys * (keys - 2.0 * eps),
                              (((1,), (1,)), ((), ())),
                              precision=hi, preferred_element_type=f32)

        f1 = (s == m1).astype(f32)
        r1 = 1.0 / jnp.sum(f1, axis=1, keepdims=True)
        kv1 = jnp.sum(f1 * g_t, axis=1, keepdims=True) * r1
        ksq1 = jnp.sum(f1 * ksq_t, axis=1, keepdims=True) * r1
        s2 = jnp.where(s == m1, -jnp.inf, s)
        m2 = jnp.max(s2, axis=1, keepdims=True)
        f2 = (s2 == m2).astype(f32)
        r2 = 1.0 / jnp.sum(f2, axis=1, keepdims=True)
        kv2 = jnp.sum(f2 * g_t, axis=1, keepdims=True) * r2

        dpos = jnp.sqrt(jnp.maximum(a - 2.0 * m1 + kv1, 0.0))
        dneg = jnp.sqrt(jnp.maximum(a - 2.0 * m2 + kv2, 0.0))
        gp = (jnp.sum(qsq_l) - 2.0 * jnp.sum(m1) + jnp.sum(ksq1))
        sp = jnp.sum(jnp.maximum(dpos - dneg + 1.0, 0.0))
        gp_part = jnp.full((1, 128), gp, f32)
        sp_part = jnp.full((1, 128), sp, f32)
        gp_s[...] = jnp.where(i == 0, gp_part, gp_s[...] + gp_part)
        sp_s[...] = jnp.where(i == 0, sp_part, sp_s[...] + sp_part)

        c_t = lax.dot_general(keys, p, (((0,), (1,)), ((), ())),
                              preferred_element_type=f32)
        w1 = w_ref[0:_D, :]
        w2 = w_ref[_D:2 * _D, :]
        conv = (lax.dot_general(w1, qn, (((0,), (0,)), ((), ())),
                                preferred_element_type=f32) +
                lax.dot_general(w2, c_t, (((0,), (0,)), ((), ())),
                                preferred_element_type=f32))
        conv_s[i] = conv
        csum = jnp.sum(conv, axis=1, keepdims=True)
        csq = jnp.sum(conv * conv, axis=1, keepdims=True)
        bnsum_s[...] = jnp.where(i == 0, csum, bnsum_s[...] + csum)
        bnsq_s[...] = jnp.where(i == 0, csq, bnsq_s[...] + csq)

    @pl.when(i >= _B)
    def _phase_b():
        kb = keysb_ref[...]
        s = jnp.concatenate(
            [lax.dot_general(qn_s[b], kb, (((0,), (1,)), ((), ())),
                             preferred_element_type=f32)
             for b in range(_B)], axis=0)
        cm = jnp.max(s, axis=0, keepdims=True)
        e = jnp.exp(s - cm)
        sq_ref[...] = e * (1.0 / jnp.sum(e, axis=0, keepdims=True))

        conv = conv_s[i - _B]
        mean = bnsum_s[...] * (1.0 / _N)
        var = bnsq_s[...] * (1.0 / _N) - mean * mean
        inv = 1.0 / jnp.sqrt(var + 1e-5)
        y = jnp.maximum((conv - mean) * inv * gamma_ref[...] + beta_ref[...],
                        0.0)
        uq_ref[...] = y.reshape(1, _D, _HW)

        gl_ref[...] = gp_s[0:1, 0:1] * (1.0 / (_N * _D))
        sl_ref[...] = sp_s[0:1, 0:1] * (1.0 / _N)


@functools.partial(jax.jit, static_argnames=())
def kernel(query, keys, W, gamma, beta):
    f32 = jnp.float32
    q3 = query.reshape(_B, _D, _HW)
    sm, sq, uq, gl, sl = pl.pallas_call(
        _fused,
        grid=(2 * _B,),
        in_specs=[
            pl.BlockSpec((1, _D, _HW), lambda i: (jnp.minimum(i, _B - 1), 0, 0)),
            pl.BlockSpec((_M, _D), lambda i: (0, 0)),
            pl.BlockSpec((_MB, _D), lambda i: (jnp.maximum(i - _B, 0), 0)),
            pl.BlockSpec((2 * _D, _D), lambda i: (0, 0)),
            pl.BlockSpec((_D, 1), lambda i: (0, 0)),
            pl.BlockSpec((_D, 1), lambda i: (0, 0)),
        ],
        out_specs=[
            pl.BlockSpec((_HW, _M), lambda i: (jnp.minimum(i, _B - 1), 0)),
            pl.BlockSpec((_N, _MB), lambda i: (0, jnp.maximum(i - _B, 0))),
            pl.BlockSpec((1, _D, _HW), lambda i: (jnp.maximum(i - _B, 0), 0, 0)),
            pl.BlockSpec((1, 1), lambda i: (0, 0)),
            pl.BlockSpec((1, 1), lambda i: (0, 0)),
        ],
        out_shape=[
            jax.ShapeDtypeStruct((_N, _M), f32),
            jax.ShapeDtypeStruct((_N, _M), f32),
            jax.ShapeDtypeStruct((_B, _D, _HW), f32),
            jax.ShapeDtypeStruct((1, 1), f32),
            jax.ShapeDtypeStruct((1, 1), f32),
        ],
        scratch_shapes=[
            pltpu.VMEM((_B, _D, _HW), f32),
            pltpu.VMEM((_B, _D, _HW), f32),
            pltpu.VMEM((_D, 1), f32),
            pltpu.VMEM((_D, 1), f32),
            pltpu.VMEM((1, 128), f32),
            pltpu.VMEM((1, 128), f32),
        ],
    )(q3, keys, keys, W, gamma.reshape(_D, 1), beta.reshape(_D, 1))

    return (uq.reshape(_B, _D, 32, 32), sq, sm, gl.reshape(()), sl.reshape(()))


# speedup vs baseline: 14.1119x; 1.0510x over previous
import functools

import jax
import jax.numpy as jnp
from jax import lax
from jax.experimental import pallas as pl
from jax.experimental.pallas import tpu as pltpu

_N = 8192
_M = 1024
_D = 256
_B = 8
_HW = 1024
_MB = 128


def _fused(q_ref, keysf_ref, keysb_ref, w_ref, gamma_ref, beta_ref,
           sm_ref, sq_ref, uq_ref, gl_ref, sl_ref,
           qn_s, conv_s, bnsum_s, bnsq_s, gp_s, sp_s, kst_s):
    f32 = jnp.float32
    i = pl.program_id(0)

    @pl.when(i == 0)
    def _key_stats():
        hi = jax.lax.Precision.HIGHEST
        eps = 1e-6
        keys = keysf_ref[...]
        ones_d = jnp.ones((1, _D), f32)
        kst_s[0:1, :] = lax.dot_general(
            ones_d, keys * keys, (((1,), (1,)), ((), ())),
            precision=hi, preferred_element_type=f32)
        kst_s[1:2, :] = lax.dot_general(
            ones_d, keys * (keys - 2.0 * eps), (((1,), (1,)), ((), ())),
            precision=hi, preferred_element_type=f32)

    @pl.when(i < _B)
    def _phase_a():
        q = q_ref[...].reshape(_D, _HW)
        n2 = jnp.sum(q * q, axis=0, keepdims=True)
        denom = jnp.maximum(jnp.sqrt(n2), 1e-12)
        qn = q / denom
        qn_s[i] = qn
        qsq_l = n2 / (denom * denom)
        qsum_l = jnp.sum(q, axis=0, keepdims=True) / denom

        eps = 1e-6
        a_l = qsq_l + (2.0 * eps) * qsum_l + _D * eps * eps
        a = jnp.transpose(a_l, (1, 0))

        keys = keysf_ref[...]
        s = lax.dot_general(qn, keys, (((0,), (1,)), ((), ())),
                            preferred_element_type=f32)

        m1 = jnp.max(s, axis=1, keepdims=True)
        e = jnp.exp(s - m1)
        p = e * (1.0 / jnp.sum(e, axis=1, keepdims=True))
        sm_ref[...] = p

        ksq_t = kst_s[0:1, :]
        g_t = kst_s[1:2, :]

        mk1 = s == m1
        r1 = 1.0 / jnp.sum(jnp.where(mk1, 1.0, 0.0), axis=1, keepdims=True)
        kv1 = jnp.sum(jnp.where(mk1, g_t, 0.0), axis=1, keepdims=True) * r1
        ksq1 = jnp.sum(jnp.where(mk1, ksq_t, 0.0), axis=1, keepdims=True) * r1
        s2 = jnp.where(mk1, -jnp.inf, s)
        m2 = jnp.max(s2, axis=1, keepdims=True)
        mk2 = s2 == m2
        r2 = 1.0 / jnp.sum(jnp.where(mk2, 1.0, 0.0), axis=1, keepdims=True)
        kv2 = jnp.sum(jnp.where(mk2, g_t, 0.0), axis=1, keepdims=True) * r2

        dpos = jnp.sqrt(jnp.maximum(a - 2.0 * m1 + kv1, 0.0))
        dneg = jnp.sqrt(jnp.maximum(a - 2.0 * m2 + kv2, 0.0))
        gp = (jnp.sum(qsq_l) - 2.0 * jnp.sum(m1) + jnp.sum(ksq1))
        sp = jnp.sum(jnp.maximum(dpos - dneg + 1.0, 0.0))
        gp_part = jnp.full((1, 128), gp, f32)
        sp_part = jnp.full((1, 128), sp, f32)
        gp_s[...] = jnp.where(i == 0, gp_part, gp_s[...] + gp_part)
        sp_s[...] = jnp.where(i == 0, sp_part, sp_s[...] + sp_part)

        c_t = lax.dot_general(keys, p, (((0,), (1,)), ((), ())),
                              preferred_element_type=f32)
        w1 = w_ref[0:_D, :]
        w2 = w_ref[_D:2 * _D, :]
        conv = (lax.dot_general(w1, qn, (((0,), (0,)), ((), ())),
                                preferred_element_type=f32) +
                lax.dot_general(w2, c_t, (((0,), (0,)), ((), ())),
                                preferred_element_type=f32))
        conv_s[i] = conv
        csum = jnp.sum(conv, axis=1, keepdims=True)
        csq = jnp.sum(conv * conv, axis=1, keepdims=True)
        bnsum_s[...] = jnp.where(i == 0, csum, bnsum_s[...] + csum)
        bnsq_s[...] = jnp.where(i == 0, csq, bnsq_s[...] + csq)

    @pl.when(i >= _B)
    def _phase_b():
        kb = keysb_ref[...]
        sb = [lax.dot_general(qn_s[b], kb, (((0,), (1,)), ((), ())),
                              preferred_element_type=f32)
              for b in range(_B)]
        cm = sb[0].max(axis=0, keepdims=True)
        for x in sb[1:]:
            cm = jnp.maximum(cm, x.max(axis=0, keepdims=True))
        eb = [jnp.exp(x - cm) for x in sb]
        cs = eb[0].sum(axis=0, keepdims=True)
        for x in eb[1:]:
            cs = cs + x.sum(axis=0, keepdims=True)
        rcs = 1.0 / cs
        for b in range(_B):
            sq_ref[b * _HW:(b + 1) * _HW, :] = eb[b] * rcs

        conv = conv_s[i - _B]
        mean = bnsum_s[...] * (1.0 / _N)
        var = bnsq_s[...] * (1.0 / _N) - mean * mean
        inv = 1.0 / jnp.sqrt(var + 1e-5)
        y = jnp.maximum((conv - mean) * inv * gamma_ref[...] + beta_ref[...],
                        0.0)
        uq_ref[...] = y.reshape(1, _D, _HW)

        gl_ref[...] = gp_s[0:1, 0:1] * (1.0 / (_N * _D))
        sl_ref[...] = sp_s[0:1, 0:1] * (1.0 / _N)


@functools.partial(jax.jit, static_argnames=())
def kernel(query, keys, W, gamma, beta):
    f32 = jnp.float32
    q3 = query.reshape(_B, _D, _HW)
    sm, sq, uq, gl, sl = pl.pallas_call(
        _fused,
        grid=(2 * _B,),
        in_specs=[
            pl.BlockSpec((1, _D, _HW), lambda i: (jnp.minimum(i, _B - 1), 0, 0)),
            pl.BlockSpec((_M, _D), lambda i: (0, 0)),
            pl.BlockSpec((_MB, _D), lambda i: (jnp.maximum(i - _B, 0), 0)),
            pl.BlockSpec((2 * _D, _D), lambda i: (0, 0)),
            pl.BlockSpec((_D, 1), lambda i: (0, 0)),
            pl.BlockSpec((_D, 1), lambda i: (0, 0)),
        ],
        out_specs=[
            pl.BlockSpec((_HW, _M), lambda i: (jnp.minimum(i, _B - 1), 0)),
            pl.BlockSpec((_N, _MB), lambda i: (0, jnp.maximum(i - _B, 0))),
            pl.BlockSpec((1, _D, _HW), lambda i: (jnp.maximum(i - _B, 0), 0, 0)),
            pl.BlockSpec((1, 1), lambda i: (0, 0)),
            pl.BlockSpec((1, 1), lambda i: (0, 0)),
        ],
        out_shape=[
            jax.ShapeDtypeStruct((_N, _M), f32),
            jax.ShapeDtypeStruct((_N, _M), f32),
            jax.ShapeDtypeStruct((_B, _D, _HW), f32),
            jax.ShapeDtypeStruct((1, 1), f32),
            jax.ShapeDtypeStruct((1, 1), f32),
        ],
        scratch_shapes=[
            pltpu.VMEM((_B, _D, _HW), f32),
            pltpu.VMEM((_B, _D, _HW), f32),
            pltpu.VMEM((_D, 1), f32),
            pltpu.VMEM((_D, 1), f32),
            pltpu.VMEM((1, 128), f32),
            pltpu.VMEM((1, 128), f32),
            pltpu.VMEM((2, _M), f32),
        ],
    )(q3, keys, keys, W, gamma.reshape(_D, 1), beta.reshape(_D, 1))

    return (uq.reshape(_B, _D, 32, 32), sq, sm, gl.reshape(()), sl.reshape(()))


# speedup vs baseline: 14.7816x; 1.0475x over previous
import functools

import jax
import jax.numpy as jnp
from jax import lax
from jax.experimental import pallas as pl
from jax.experimental.pallas import tpu as pltpu

_N = 8192
_M = 1024
_D = 256
_B = 8
_HW = 1024
_MB = 256
_BSTEPS = _M // _MB


def _fused(q_ref, keysf_ref, keysb_ref, w_ref, gamma_ref, beta_ref,
           sm_ref, sq_ref, uq_ref, gl_ref, sl_ref,
           qn_s, conv_s, bnsum_s, bnsq_s, gp_s, sp_s, kst_s):
    f32 = jnp.float32
    i = pl.program_id(0)

    @pl.when(i == 0)
    def _key_stats():
        hi = jax.lax.Precision.HIGHEST
        eps = 1e-6
        keys = keysf_ref[...]
        ones_d = jnp.ones((1, _D), f32)
        kst_s[0:1, :] = lax.dot_general(
            ones_d, keys * keys, (((1,), (1,)), ((), ())),
            precision=hi, preferred_element_type=f32)
        kst_s[1:2, :] = lax.dot_general(
            ones_d, keys * (keys - 2.0 * eps), (((1,), (1,)), ((), ())),
            precision=hi, preferred_element_type=f32)

    @pl.when(i < _B)
    def _phase_a():
        q = q_ref[...].reshape(_D, _HW)
        n2 = jnp.sum(q * q, axis=0, keepdims=True)
        denom = jnp.maximum(jnp.sqrt(n2), 1e-12)
        qn = q / denom
        qn_s[i] = qn
        qsq_l = n2 / (denom * denom)
        qsum_l = jnp.sum(q, axis=0, keepdims=True) / denom

        eps = 1e-6
        a_l = qsq_l + (2.0 * eps) * qsum_l + _D * eps * eps
        a = jnp.transpose(a_l, (1, 0))

        keys = keysf_ref[...]
        s = lax.dot_general(qn, keys, (((0,), (1,)), ((), ())),
                            preferred_element_type=f32)

        m1 = jnp.max(s, axis=1, keepdims=True)
        e = jnp.exp(s - m1)
        p = e * (1.0 / jnp.sum(e, axis=1, keepdims=True))
        sm_ref[...] = p

        ksq_t = kst_s[0:1, :]
        g_t = kst_s[1:2, :]

        mk1 = s == m1
        r1 = 1.0 / jnp.sum(jnp.where(mk1, 1.0, 0.0), axis=1, keepdims=True)
        kv1 = jnp.sum(jnp.where(mk1, g_t, 0.0), axis=1, keepdims=True) * r1
        ksq1 = jnp.sum(jnp.where(mk1, ksq_t, 0.0), axis=1, keepdims=True) * r1
        s2 = jnp.where(mk1, -jnp.inf, s)
        m2 = jnp.max(s2, axis=1, keepdims=True)
        mk2 = s2 == m2
        r2 = 1.0 / jnp.sum(jnp.where(mk2, 1.0, 0.0), axis=1, keepdims=True)
        kv2 = jnp.sum(jnp.where(mk2, g_t, 0.0), axis=1, keepdims=True) * r2

        dpos = jnp.sqrt(jnp.maximum(a - 2.0 * m1 + kv1, 0.0))
        dneg = jnp.sqrt(jnp.maximum(a - 2.0 * m2 + kv2, 0.0))
        gp = (jnp.sum(qsq_l) - 2.0 * jnp.sum(m1) + jnp.sum(ksq1))
        sp = jnp.sum(jnp.maximum(dpos - dneg + 1.0, 0.0))
        gp_part = jnp.full((1, 128), gp, f32)
        sp_part = jnp.full((1, 128), sp, f32)
        gp_s[...] = jnp.where(i == 0, gp_part, gp_s[...] + gp_part)
        sp_s[...] = jnp.where(i == 0, sp_part, sp_s[...] + sp_part)

        c_t = lax.dot_general(keys, p, (((0,), (1,)), ((), ())),
                              preferred_element_type=f32)
        w1 = w_ref[0:_D, :]
        w2 = w_ref[_D:2 * _D, :]
        conv = (lax.dot_general(w1, qn, (((0,), (0,)), ((), ())),
                                preferred_element_type=f32) +
                lax.dot_general(w2, c_t, (((0,), (0,)), ((), ())),
                                preferred_element_type=f32))
        conv_s[i] = conv
        csum = jnp.sum(conv, axis=1, keepdims=True)
        csq = jnp.sum(conv * conv, axis=1, keepdims=True)
        bnsum_s[...] = jnp.where(i == 0, csum, bnsum_s[...] + csum)
        bnsq_s[...] = jnp.where(i == 0, csq, bnsq_s[...] + csq)

    @pl.when(i >= _B)
    def _phase_b():
        kb = keysb_ref[...]
        sb = [lax.dot_general(qn_s[b], kb, (((0,), (1,)), ((), ())),
                              preferred_element_type=f32)
              for b in range(_B)]
        cm = sb[0].max(axis=0, keepdims=True)
        for x in sb[1:]:
            cm = jnp.maximum(cm, x.max(axis=0, keepdims=True))
        eb = [jnp.exp(x - cm) for x in sb]
        cs = eb[0].sum(axis=0, keepdims=True)
        for x in eb[1:]:
            cs = cs + x.sum(axis=0, keepdims=True)
        rcs = 1.0 / cs
        for b in range(_B):
            sq_ref[b * _HW:(b + 1) * _HW, :] = eb[b] * rcs

        mean = bnsum_s[...] * (1.0 / _N)
        var = bnsq_s[...] * (1.0 / _N) - mean * mean
        inv = 1.0 / jnp.sqrt(var + 1e-5)
        scale = inv * gamma_ref[...]
        for t in range(2):
            conv = conv_s[2 * (i - _B) + t]
            y = jnp.maximum((conv - mean) * scale + beta_ref[...], 0.0)
            uq_ref[t:t + 1] = y.reshape(1, _D, _HW)

        gl_ref[...] = gp_s[0:1, 0:1] * (1.0 / (_N * _D))
        sl_ref[...] = sp_s[0:1, 0:1] * (1.0 / _N)


@functools.partial(jax.jit, static_argnames=())
def kernel(query, keys, W, gamma, beta):
    f32 = jnp.float32
    q3 = query.reshape(_B, _D, _HW)
    sm, sq, uq, gl, sl = pl.pallas_call(
        _fused,
        grid=(_B + _BSTEPS,),
        in_specs=[
            pl.BlockSpec((1, _D, _HW), lambda i: (jnp.minimum(i, _B - 1), 0, 0)),
            pl.BlockSpec((_M, _D), lambda i: (0, 0)),
            pl.BlockSpec((_MB, _D), lambda i: (jnp.maximum(i - _B, 0), 0)),
            pl.BlockSpec((2 * _D, _D), lambda i: (0, 0)),
            pl.BlockSpec((_D, 1), lambda i: (0, 0)),
            pl.BlockSpec((_D, 1), lambda i: (0, 0)),
        ],
        out_specs=[
            pl.BlockSpec((_HW, _M), lambda i: (jnp.minimum(i, _B - 1), 0)),
            pl.BlockSpec((_N, _MB), lambda i: (0, jnp.maximum(i - _B, 0))),
            pl.BlockSpec((2, _D, _HW), lambda i: (jnp.maximum(i - _B, 0), 0, 0)),
            pl.BlockSpec((1, 1), lambda i: (0, 0)),
            pl.BlockSpec((1, 1), lambda i: (0, 0)),
        ],
        out_shape=[
            jax.ShapeDtypeStruct((_N, _M), f32),
            jax.ShapeDtypeStruct((_N, _M), f32),
            jax.ShapeDtypeStruct((_B, _D, _HW), f32),
            jax.ShapeDtypeStruct((1, 1), f32),
            jax.ShapeDtypeStruct((1, 1), f32),
        ],
        scratch_shapes=[
            pltpu.VMEM((_B, _D, _HW), f32),
            pltpu.VMEM((_B, _D, _HW), f32),
            pltpu.VMEM((_D, 1), f32),
            pltpu.VMEM((_D, 1), f32),
            pltpu.VMEM((1, 128), f32),
            pltpu.VMEM((1, 128), f32),
            pltpu.VMEM((2, _M), f32),
        ],
    )(q3, keys, keys, W, gamma.reshape(_D, 1), beta.reshape(_D, 1))

    return (uq.reshape(_B, _D, 32, 32), sq, sm, gl.reshape(()), sl.reshape(()))


# speedup vs baseline: 16.7991x; 1.1365x over previous
import functools

import jax
import jax.numpy as jnp
from jax import lax
from jax.experimental import pallas as pl
from jax.experimental.pallas import tpu as pltpu

_N = 8192
_M = 1024
_D = 256
_B = 8
_HW = 1024
_MB = 256
_BSTEPS = _M // _MB


def _fused(q_ref, keysf_ref, keysb_ref, w_ref, gamma_ref, beta_ref,
           sm_ref, sq_ref, uq_ref, gl_ref, sl_ref,
           qn_s, conv_s, bnsum_s, bnsq_s, gp_s, sp_s, kst_s):
    f32 = jnp.float32
    i = pl.program_id(0)

    @pl.when(i == 0)
    def _key_stats():
        hi = jax.lax.Precision.HIGHEST
        eps = 1e-6
        keys = keysf_ref[...]
        ones_d = jnp.ones((1, _D), f32)
        kst_s[0:1, :] = lax.dot_general(
            ones_d, keys * (keys - 2.0 * eps), (((1,), (1,)), ((), ())),
            precision=hi, preferred_element_type=f32)

    @pl.when(i < _B)
    def _phase_a():
        q = q_ref[...].reshape(_D, _HW)
        n2 = jnp.sum(q * q, axis=0, keepdims=True)
        denom = jnp.maximum(jnp.sqrt(n2), 1e-12)
        qn = q / denom
        qn_s[i] = qn
        qsq_l = n2 / (denom * denom)
        qsum_l = jnp.sum(q, axis=0, keepdims=True) / denom

        eps = 1e-6
        a_l = qsq_l + (2.0 * eps) * qsum_l + _D * eps * eps
        a = jnp.transpose(a_l, (1, 0))

        keys = keysf_ref[...]
        s = lax.dot_general(qn, keys, (((0,), (1,)), ((), ())),
                            preferred_element_type=f32)

        m1 = jnp.max(s, axis=1, keepdims=True)
        e = jnp.exp(s - m1)
        p = e * (1.0 / jnp.sum(e, axis=1, keepdims=True))
        sm_ref[...] = p

        g_t = kst_s[0:1, :]

        mk1 = s == m1
        kv1 = jnp.max(jnp.where(mk1, g_t, -jnp.inf), axis=1, keepdims=True)
        s2 = jnp.where(mk1, -jnp.inf, s)
        m2 = jnp.max(s2, axis=1, keepdims=True)
        kv2 = jnp.max(jnp.where(s2 == m2, g_t, -jnp.inf), axis=1,
                      keepdims=True)

        dpos2 = jnp.maximum(a - 2.0 * m1 + kv1, 0.0)
        dpos = jnp.sqrt(dpos2)
        dneg = jnp.sqrt(jnp.maximum(a - 2.0 * m2 + kv2, 0.0))
        gp = jnp.sum(dpos2)
        sp = jnp.sum(jnp.maximum(dpos - dneg + 1.0, 0.0))
        gp_part = jnp.full((1, 128), gp, f32)
        sp_part = jnp.full((1, 128), sp, f32)
        gp_s[...] = jnp.where(i == 0, gp_part, gp_s[...] + gp_part)
        sp_s[...] = jnp.where(i == 0, sp_part, sp_s[...] + sp_part)

        c_t = lax.dot_general(keys, p, (((0,), (1,)), ((), ())),
                              preferred_element_type=f32)
        w1 = w_ref[0:_D, :]
        w2 = w_ref[_D:2 * _D, :]
        conv = (lax.dot_general(w1, qn, (((0,), (0,)), ((), ())),
                                preferred_element_type=f32) +
                lax.dot_general(w2, c_t, (((0,), (0,)), ((), ())),
                                preferred_element_type=f32))
        conv_s[i] = conv
        csum = jnp.sum(conv, axis=1, keepdims=True)
        csq = jnp.sum(conv * conv, axis=1, keepdims=True)
        bnsum_s[...] = jnp.where(i == 0, csum, bnsum_s[...] + csum)
        bnsq_s[...] = jnp.where(i == 0, csq, bnsq_s[...] + csq)

    @pl.when(i >= _B)
    def _phase_b():
        kb = keysb_ref[...]
        sb = [lax.dot_general(qn_s[b], kb, (((0,), (1,)), ((), ())),
                              preferred_element_type=f32)
              for b in range(_B)]
        cm = sb[0].max(axis=0, keepdims=True)
        for x in sb[1:]:
            cm = jnp.maximum(cm, x.max(axis=0, keepdims=True))
        eb = [jnp.exp(x - cm) for x in sb]
        cs = eb[0].sum(axis=0, keepdims=True)
        for x in eb[1:]:
            cs = cs + x.sum(axis=0, keepdims=True)
        rcs = 1.0 / cs
        for b in range(_B):
            sq_ref[b * _HW:(b + 1) * _HW, :] = eb[b] * rcs

        mean = bnsum_s[...] * (1.0 / _N)
        var = bnsq_s[...] * (1.0 / _N) - mean * mean
        inv = 1.0 / jnp.sqrt(var + 1e-5)
        scale = inv * gamma_ref[...]
        for t in range(2):
            conv = conv_s[2 * (i - _B) + t]
            y = jnp.maximum((conv - mean) * scale + beta_ref[...], 0.0)
            uq_ref[t:t + 1] = y.reshape(1, _D, _HW)

        gl_ref[...] = gp_s[0:1, 0:1] * (1.0 / (_N * _D))
        sl_ref[...] = sp_s[0:1, 0:1] * (1.0 / _N)


@functools.partial(jax.jit, static_argnames=())
def kernel(query, keys, W, gamma, beta):
    f32 = jnp.float32
    q3 = query.reshape(_B, _D, _HW)
    sm, sq, uq, gl, sl = pl.pallas_call(
        _fused,
        grid=(_B + _BSTEPS,),
        in_specs=[
            pl.BlockSpec((1, _D, _HW), lambda i: (jnp.minimum(i, _B - 1), 0, 0)),
            pl.BlockSpec((_M, _D), lambda i: (0, 0)),
            pl.BlockSpec((_MB, _D), lambda i: (jnp.maximum(i - _B, 0), 0)),
            pl.BlockSpec((2 * _D, _D), lambda i: (0, 0)),
            pl.BlockSpec((_D, 1), lambda i: (0, 0)),
            pl.BlockSpec((_D, 1), lambda i: (0, 0)),
        ],
        out_specs=[
            pl.BlockSpec((_HW, _M), lambda i: (jnp.minimum(i, _B - 1), 0)),
            pl.BlockSpec((_N, _MB), lambda i: (0, jnp.maximum(i - _B, 0))),
            pl.BlockSpec((2, _D, _HW), lambda i: (jnp.maximum(i - _B, 0), 0, 0)),
            pl.BlockSpec((1, 1), lambda i: (0, 0)),
            pl.BlockSpec((1, 1), lambda i: (0, 0)),
        ],
        out_shape=[
            jax.ShapeDtypeStruct((_N, _M), f32),
            jax.ShapeDtypeStruct((_N, _M), f32),
            jax.ShapeDtypeStruct((_B, _D, _HW), f32),
            jax.ShapeDtypeStruct((1, 1), f32),
            jax.ShapeDtypeStruct((1, 1), f32),
        ],
        scratch_shapes=[
            pltpu.VMEM((_B, _D, _HW), f32),
            pltpu.VMEM((_B, _D, _HW), f32),
            pltpu.VMEM((_D, 1), f32),
            pltpu.VMEM((_D, 1), f32),
            pltpu.VMEM((1, 128), f32),
            pltpu.VMEM((1, 128), f32),
            pltpu.VMEM((1, _M), f32),
        ],
    )(q3, keys, keys, W, gamma.reshape(_D, 1), beta.reshape(_D, 1))

    return (uq.reshape(_B, _D, 32, 32), sq, sm, gl.reshape(()), sl.reshape(()))
